# Initial kernel scaffold; baseline (speedup 1.0000x reference)
import jax
import jax.numpy as jnp
from jax.experimental import pallas as pl


def kernel(x, params, edge_index, batch, train):
    raise NotImplementedError("write your pallas kernel here")


# speedup vs baseline: 9.0034x; 9.0034x over previous
import functools

import jax
import jax.numpy as jnp
from jax import lax
from jax.experimental import pallas as pl
from jax.experimental.pallas import tpu as pltpu
from jax.experimental.pallas import tpu_sc as plsc

N = 10000
NUM_GRAPHS = 64
NR = 10240
NRS = 10016
LANES = 16
NC, NS = 2, 16
NW = NC * NS
C = 128
BLK = 512
MBLK = 1024


def _edge_sc(xl, xr, att2d, s_idx, d_idx, d, k_chunks):
    nj = d // LANES
    nch, rem = divmod(NRS, C)
    mesh = plsc.VectorSubcoreMesh(core_axis_name="c", subcore_axis_name="s")

    @functools.partial(
        pl.kernel,
        out_type=[
            jax.ShapeDtypeStruct((NC, NR, d), jnp.float32),
            jax.ShapeDtypeStruct((NC, NR, LANES), jnp.float32),
        ],
        mesh=mesh,
        compiler_params=pltpu.CompilerParams(use_tc_tiling_on_sc=False),
        scratch_types=[
            pltpu.VMEM((C,), jnp.int32),
            pltpu.VMEM((C,), jnp.int32),
            pltpu.VMEM((C, d), jnp.float32),
            pltpu.VMEM((C, d), jnp.float32),
            pltpu.VMEM((C, LANES), jnp.float32),
            pltpu.VMEM((nj, LANES), jnp.float32),
            pltpu.VMEM_SHARED((NRS, d), jnp.float32),
            pltpu.VMEM_SHARED((NRS, LANES), jnp.float32),
            pltpu.SemaphoreType.DMA,
        ],
    )
    def ek(xl_hbm, xr_hbm, att_hbm, s_hbm, dd_hbm, num_out, den_out,
           sidx, didx, xlr, xrr, exb, attv, num_sh, den_sh, sem):
        cid = lax.axis_index("c")
        sid = lax.axis_index("s")
        wid = cid * NS + sid

        pltpu.sync_copy(att_hbm, attv)

        def zrow(i, _):
            for j in range(nj):
                xlr[i, pl.ds(j * LANES, LANES)] = jnp.zeros((LANES,), jnp.float32)
            exb[i, :] = jnp.zeros((LANES,), jnp.float32)
            return 0
        lax.fori_loop(0, C, zrow, 0)
        def zchunk(c, _):
            @pl.when(c % NS == sid)
            def _():
                off = pl.multiple_of(c * C, C)
                pltpu.sync_copy(xlr, num_sh.at[pl.ds(off, C)])
                pltpu.sync_copy(exb, den_sh.at[pl.ds(off, C)])
            return 0
        lax.fori_loop(0, nch, zchunk, 0)
        if rem:
            @pl.when(sid == nch % NS)
            def _():
                pltpu.sync_copy(xlr.at[pl.ds(0, rem)],
                                num_sh.at[pl.ds(nch * C, rem)])
                pltpu.sync_copy(exb.at[pl.ds(0, rem)],
                                den_sh.at[pl.ds(nch * C, rem)])
        plsc.subcore_barrier()

        base = wid * (k_chunks * C)

        def chunk(k, _):
            off = pl.multiple_of(base + k * C, C)
            pltpu.sync_copy(s_hbm.at[pl.ds(off, C)], sidx)
            pltpu.sync_copy(dd_hbm.at[pl.ds(off, C)], didx)
            pltpu.async_copy(xl_hbm.at[sidx], xlr, sem).wait()
            pltpu.async_copy(xr_hbm.at[didx], xrr, sem).wait()

            def edge(i, _):
                acc = jnp.zeros((LANES,), jnp.float32)
                for j in range(nj):
                    sl = pl.ds(j * LANES, LANES)
                    z = xlr[i, sl] + xrr[i, sl]
                    z = jnp.maximum(z, 0.2 * z)
                    acc = acc + z * attv[j, :]
                lane = lax.iota(jnp.int32, LANES)
                dn = lax.GatherDimensionNumbers(
                    offset_dims=(), collapsed_slice_dims=(0,),
                    start_index_map=(0,))
                for st in (8, 4, 2, 1):
                    acc = acc + lax.gather(
                        acc, (lane ^ st)[:, None], dn, slice_sizes=(1,),
                        mode=lax.GatherScatterMode.PROMISE_IN_BOUNDS)
                ex = jnp.exp(acc)
                exb[i, :] = ex
                for j in range(nj):
                    sl = pl.ds(j * LANES, LANES)
                    xlr[i, sl] = xlr[i, sl] * ex
                return 0
            lax.fori_loop(0, C, edge, 0)

            pltpu.sync_copy(xlr, num_sh.at[didx], add=True)
            pltpu.sync_copy(exb, den_sh.at[didx], add=True)
            return 0
        lax.fori_loop(0, k_chunks, chunk, 0)

        plsc.subcore_barrier()

        def dchunk(c, _):
            @pl.when(c % NS == sid)
            def _():
                off = pl.multiple_of(c * C, C)
                pltpu.sync_copy(num_sh.at[pl.ds(off, C)],
                                num_out.at[cid, pl.ds(off, C)])
                pltpu.sync_copy(den_sh.at[pl.ds(off, C)],
                                den_out.at[cid, pl.ds(off, C)])
            return 0
        lax.fori_loop(0, nch, dchunk, 0)
        if rem:
            @pl.when(sid == nch % NS)
            def _():
                pltpu.sync_copy(num_sh.at[pl.ds(nch * C, rem)],
                                num_out.at[cid, pl.ds(nch * C, rem)])
                pltpu.sync_copy(den_sh.at[pl.ds(nch * C, rem)],
                                den_out.at[cid, pl.ds(nch * C, rem)])

    return ek(xl, xr, att2d, s_idx, d_idx)


def _mm2_body(h_ref, wl_ref, wr_ref, xl_ref, xr_ref):
    h = h_ref[...]
    xl_ref[...] = jnp.dot(h, wl_ref[...], preferred_element_type=jnp.float32)
    xr_ref[...] = jnp.dot(h, wr_ref[...], preferred_element_type=jnp.float32)


def _mm2(h, wl, wr):
    din, dout = wl.shape
    return pl.pallas_call(
        _mm2_body,
        grid=(NR // BLK,),
        in_specs=[
            pl.BlockSpec((BLK, din), lambda i: (i, 0)),
            pl.BlockSpec((din, dout), lambda i: (0, 0)),
            pl.BlockSpec((din, dout), lambda i: (0, 0)),
        ],
        out_specs=[
            pl.BlockSpec((BLK, dout), lambda i: (i, 0)),
            pl.BlockSpec((BLK, dout), lambda i: (i, 0)),
        ],
        out_shape=[
            jax.ShapeDtypeStruct((NR, dout), jnp.float32),
            jax.ShapeDtypeStruct((NR, dout), jnp.float32),
        ],
    )(h, wl, wr)


def _merge_body(num_ref, den_ref, b_ref, h_ref, sums_ref):
    i = pl.program_id(0)
    d = num_ref.shape[2]
    dsum = den_ref[0, :, 0] + den_ref[1, :, 0]
    out = (num_ref[0] + num_ref[1]) / (dsum[:, None] + 1e-16) + b_ref[0, :][None, :]
    out = jnp.maximum(out, 0.0)
    row = i * MBLK + lax.broadcasted_iota(jnp.int32, (MBLK, 1), 0)
    out = jnp.where(row < N, out, 0.0)
    h_ref[...] = out

    @pl.when(i == 0)
    def _():
        sums_ref[...] = jnp.zeros_like(sums_ref)
    s1 = jnp.sum(out, axis=0)
    s2 = jnp.sum(out * out, axis=0)
    sums_ref[...] += jnp.concatenate(
        [s1[None], s2[None], jnp.zeros((6, d), jnp.float32)], axis=0)


def _merge(num, den, bias):
    d = num.shape[2]
    return pl.pallas_call(
        _merge_body,
        grid=(NR // MBLK,),
        in_specs=[
            pl.BlockSpec((2, MBLK, d), lambda i: (0, i, 0)),
            pl.BlockSpec((2, MBLK, LANES), lambda i: (0, i, 0)),
            pl.BlockSpec((1, d), lambda i: (0, 0)),
        ],
        out_specs=[
            pl.BlockSpec((MBLK, d), lambda i: (i, 0)),
            pl.BlockSpec((8, d), lambda i: (0, 0)),
        ],
        out_shape=[
            jax.ShapeDtypeStruct((NR, d), jnp.float32),
            jax.ShapeDtypeStruct((8, d), jnp.float32),
        ],
    )(num, den, bias)


def _bn_from_sums(h, sums_ref, gam_ref, bet_ref):
    mu = sums_ref[0, :] / N
    var = sums_ref[1, :] / N - mu * mu
    inv = lax.rsqrt(var + 1e-5)
    return (h - mu[None, :]) * inv[None, :] * gam_ref[0, :][None, :] \
        + bet_ref[0, :][None, :]


def _pool_contrib(b3_ref, hn, blk):
    b = b3_ref[0, 0, :]
    oh = (b[:, None] == lax.broadcasted_iota(jnp.int32, (blk, NUM_GRAPHS), 1))
    oh = oh.astype(jnp.float32)
    return lax.dot_general(oh, hn, (((0,), (0,)), ((), ())),
                           preferred_element_type=jnp.float32)


def _prep_body(h_ref, sums_ref, gam_ref, bet_ref, wl_ref, wr_ref, b3_ref,
               xl_ref, xr_ref, g_ref):
    i = pl.program_id(0)
    hn = _bn_from_sums(h_ref[...], sums_ref, gam_ref, bet_ref)
    xl_ref[...] = jnp.dot(hn, wl_ref[...], preferred_element_type=jnp.float32)
    xr_ref[...] = jnp.dot(hn, wr_ref[...], preferred_element_type=jnp.float32)

    @pl.when(i == 0)
    def _():
        g_ref[...] = jnp.zeros_like(g_ref)
    g_ref[...] += _pool_contrib(b3_ref, hn, BLK)


def _prep(h, sums, gam, bet, wl, wr, batch3):
    din, dout = wl.shape
    return pl.pallas_call(
        _prep_body,
        grid=(NR // BLK,),
        in_specs=[
            pl.BlockSpec((BLK, din), lambda i: (i, 0)),
            pl.BlockSpec((8, din), lambda i: (0, 0)),
            pl.BlockSpec((1, din), lambda i: (0, 0)),
            pl.BlockSpec((1, din), lambda i: (0, 0)),
            pl.BlockSpec((din, dout), lambda i: (0, 0)),
            pl.BlockSpec((din, dout), lambda i: (0, 0)),
            pl.BlockSpec((1, 1, BLK), lambda i: (i, 0, 0)),
        ],
        out_specs=[
            pl.BlockSpec((BLK, dout), lambda i: (i, 0)),
            pl.BlockSpec((BLK, dout), lambda i: (i, 0)),
            pl.BlockSpec((NUM_GRAPHS, din), lambda i: (0, 0)),
        ],
        out_shape=[
            jax.ShapeDtypeStruct((NR, dout), jnp.float32),
            jax.ShapeDtypeStruct((NR, dout), jnp.float32),
            jax.ShapeDtypeStruct((NUM_GRAPHS, din), jnp.float32),
        ],
    )(h, sums, gam, bet, wl, wr, batch3)


def _pool_body(h_ref, sums_ref, gam_ref, bet_ref, b3_ref, g_ref):
    i = pl.program_id(0)
    hn = _bn_from_sums(h_ref[...], sums_ref, gam_ref, bet_ref)

    @pl.when(i == 0)
    def _():
        g_ref[...] = jnp.zeros_like(g_ref)
    g_ref[...] += _pool_contrib(b3_ref, hn, BLK)


def _pool(h, sums, gam, bet, batch3):
    din = h.shape[1]
    return pl.pallas_call(
        _pool_body,
        grid=(NR // BLK,),
        in_specs=[
            pl.BlockSpec((BLK, din), lambda i: (i, 0)),
            pl.BlockSpec((8, din), lambda i: (0, 0)),
            pl.BlockSpec((1, din), lambda i: (0, 0)),
            pl.BlockSpec((1, din), lambda i: (0, 0)),
            pl.BlockSpec((1, 1, BLK), lambda i: (i, 0, 0)),
        ],
        out_specs=pl.BlockSpec((NUM_GRAPHS, din), lambda i: (0, 0)),
        out_shape=jax.ShapeDtypeStruct((NUM_GRAPHS, din), jnp.float32),
    )(h, sums, gam, bet, batch3)


def _head_body(g1_ref, g2_ref, g3_ref, w1_ref, b1_ref, gam_ref, bet_ref,
               w2_ref, b2_ref, sig_ref, lsm_ref):
    g3 = g3_ref[...]
    hcat = jnp.concatenate([g1_ref[...], g2_ref[...], g3, g3], axis=1)
    h = jnp.dot(hcat, w1_ref[...], preferred_element_type=jnp.float32) \
        + b1_ref[0, :][None, :]
    h = jnp.maximum(h, 0.0)
    mu = jnp.mean(h, axis=0)
    var = jnp.mean((h - mu[None, :]) ** 2, axis=0)
    hb = (h - mu[None, :]) / jnp.sqrt(var + 1e-5) * gam_ref[0, :][None, :] \
        + bet_ref[0, :][None, :]
    logits = jnp.dot(hb, w2_ref[...], preferred_element_type=jnp.float32) \
        + b2_ref[0, :][None, :]
    sig_ref[...] = 1.0 / (1.0 + jnp.exp(-logits))
    col = lax.broadcasted_iota(jnp.int32, logits.shape, 1)
    mask = col < 10
    ml = jnp.max(jnp.where(mask, logits, -1e30), axis=1, keepdims=True)
    e = jnp.where(mask, jnp.exp(logits - ml), 0.0)
    lse = ml + jnp.log(jnp.sum(e, axis=1, keepdims=True))
    lsm_ref[...] = logits - lse


def _head(g1, g2, g3, w1, b1, gam, bet, w2p, b2p):
    full = lambda shp: pl.BlockSpec(shp, lambda: tuple(0 for _ in shp))
    return pl.pallas_call(
        _head_body,
        in_specs=[
            full((NUM_GRAPHS, 128)), full((NUM_GRAPHS, 64)),
            full((NUM_GRAPHS, 32)),
            full((256, 128)), full((1, 128)), full((1, 128)), full((1, 128)),
            full((128, 128)), full((1, 128)),
        ],
        out_specs=[full((NUM_GRAPHS, 128)), full((NUM_GRAPHS, 128))],
        out_shape=[
            jax.ShapeDtypeStruct((NUM_GRAPHS, 128), jnp.float32),
            jax.ShapeDtypeStruct((NUM_GRAPHS, 128), jnp.float32),
        ],
    )(g1, g2, g3, w1, b1, gam, bet, w2p, b2p)


def kernel(x, params, edge_index, batch, train):
    p = params
    e = edge_index.shape[1]
    eall = e + N
    span = NW * C
    k_chunks = -(-eall // span)
    epad = k_chunks * span

    src = edge_index[0]
    dst = edge_index[1]
    loop = jnp.arange(N, dtype=jnp.int32)
    padi = jnp.full((epad - eall,), N, dtype=jnp.int32)
    s_all = jnp.concatenate([src, loop, padi])
    d_all = jnp.concatenate([dst, loop, padi])

    x_pad = jnp.zeros((NR, x.shape[1]), jnp.float32).at[:N].set(x)
    batch3 = jnp.full((NR,), NUM_GRAPHS, jnp.int32).at[:N].set(batch)
    batch3 = batch3.reshape(NR // BLK, 1, BLK)

    r1 = lambda a: a.reshape(1, -1)
    att2 = lambda a: a.reshape(-1, LANES)

    xl1, xr1 = _mm2(x_pad, p['gat1']['Wl'], p['gat1']['Wr'])
    num1, den1 = _edge_sc(xl1, xr1, att2(p['gat1']['att']), s_all, d_all,
                          128, k_chunks)
    h1, sums1 = _merge(num1, den1, r1(p['gat1']['b']))

    xl2, xr2, g1 = _prep(h1, sums1, r1(p['bn1_g']), r1(p['bn1_b']),
                         p['gat2']['Wl'], p['gat2']['Wr'], batch3)
    num2, den2 = _edge_sc(xl2, xr2, att2(p['gat2']['att']), s_all, d_all,
                          64, k_chunks)
    h2, sums2 = _merge(num2, den2, r1(p['gat2']['b']))

    xl3, xr3, g2 = _prep(h2, sums2, r1(p['bn2_g']), r1(p['bn2_b']),
                         p['gat3']['Wl'], p['gat3']['Wr'], batch3)
    num3, den3 = _edge_sc(xl3, xr3, att2(p['gat3']['att']), s_all, d_all,
                          32, k_chunks)
    h3, sums3 = _merge(num3, den3, r1(p['gat3']['b']))

    g3 = _pool(h3, sums3, r1(p['bn3_g']), r1(p['bn3_b']), batch3)

    w2p = jnp.zeros((128, 128), jnp.float32).at[:, :10].set(p['lin2W'])
    b2p = jnp.zeros((1, 128), jnp.float32).at[0, :10].set(p['lin2b'])
    sig, lsm = _head(g1, g2, g3, p['lin1W'], r1(p['lin1b']),
                     r1(p['bn5_g']), r1(p['bn5_b']), w2p, b2p)
    return sig[:, :10], lsm[:, :10]
